# baseline (speedup 1.0000x reference)
import functools

import jax
import jax.numpy as jnp
from jax import lax
from jax.experimental import pallas as pl
from jax.experimental.pallas import tpu as pltpu
from jax.experimental.pallas import tpu_sc as plsc

EPS = 1e-5

_NW = 32
_CH = 32


def _sc_gather(utab, uid, B, b_per_w):
    mesh = plsc.VectorSubcoreMesh(core_axis_name="c", subcore_axis_name="s")

    @functools.partial(
        pl.kernel,
        mesh=mesh,
        compiler_params=pltpu.CompilerParams(needs_layout_passes=False),
        out_type=jax.ShapeDtypeStruct((B * 64,), jnp.float32),
        scratch_types=[
            pltpu.VMEM((b_per_w,), jnp.int32),
            pltpu.VMEM((b_per_w * 64,), jnp.float32),
            pltpu.SemaphoreType.DMA,
        ],
    )
    def k(tab, uid_h, out, uid_v, rows_v, sem):
        wid = lax.axis_index("s") * 2 + lax.axis_index("c")
        base = wid * b_per_w
        pltpu.sync_copy(uid_h.at[pl.ds(base, b_per_w)], uid_v)

        lanes = lax.iota(jnp.int32, 16)

        def body(g, carry):
            v = uid_v[pl.ds(g * 16, 16)] * 64
            dst0 = pl.multiple_of(g * 1024, 64)
            for l in range(16):
                woff = pl.multiple_of(jnp.sum(jnp.where(lanes == l, v, 0)), 64)
                pltpu.async_copy(
                    tab.at[pl.ds(woff, 64)],
                    rows_v.at[pl.ds(dst0 + l * 64, 64)], sem)
            return carry

        lax.fori_loop(0, b_per_w // 16, body, 0)
        pltpu.make_async_copy(tab.at[pl.ds(0, b_per_w * 64)],
                              rows_v, sem).wait()
        pltpu.sync_copy(rows_v, out.at[pl.ds(base * 64, b_per_w * 64)])

    return k(utab, uid)


def _mlp_body(u_ref, c_ref, lid_ref, ltab_ref,
              w1u, w1l, w1c, b1, w2, b2, w3, b3, o_ref):
    h = jnp.dot(u_ref[...], w1u[...], preferred_element_type=jnp.float32)
    lp = jnp.dot(ltab_ref[...], w1l[...], preferred_element_type=jnp.float32)
    oh = (lid_ref[...] == lax.broadcasted_iota(
        jnp.int32, (lid_ref.shape[0], ltab_ref.shape[0]), 1)).astype(jnp.float32)
    h = h + jnp.dot(oh, lp, preferred_element_type=jnp.float32)
    h = h + jnp.dot(c_ref[...], w1c[...], preferred_element_type=jnp.float32)
    h = jnp.maximum(h + b1[...], 0.0)
    h = jnp.dot(h, w2[...], preferred_element_type=jnp.float32)
    h = jnp.maximum(h + b2[...], 0.0)
    o_ref[...] = jnp.dot(h, w3[...], preferred_element_type=jnp.float32) + b3[...]


def _mlp(u_emb, cont, lid2, lang_table, W1u, W1l, W1c, b1f, W2f, b2f, W3, b3,
         TB=2048):
    B = u_emb.shape[0]
    NL = lang_table.shape[0]
    grid = (B // TB,)
    full = lambda i: (0, 0)
    return pl.pallas_call(
        _mlp_body,
        grid=grid,
        in_specs=[
            pl.BlockSpec((TB, 64), lambda i: (i, 0)),
            pl.BlockSpec((TB, 3), lambda i: (i, 0)),
            pl.BlockSpec((TB, 1), lambda i: (i, 0)),
            pl.BlockSpec((NL, 16), full),
            pl.BlockSpec((64, 256), full),
            pl.BlockSpec((16, 256), full),
            pl.BlockSpec((3, 256), full),
            pl.BlockSpec((1, 256), full),
            pl.BlockSpec((256, 128), full),
            pl.BlockSpec((1, 128), full),
            pl.BlockSpec((128, 128), full),
            pl.BlockSpec((1, 128), full),
        ],
        out_specs=pl.BlockSpec((TB, 128), lambda i: (i, 0)),
        out_shape=jax.ShapeDtypeStruct((B, 128), jnp.float32),
    )(u_emb, cont, lid2, lang_table, W1u, W1l, W1c, b1f, W2f, b2f, W3, b3)


def kernel(user_id, user_continuous, user_lang, user_table, lang_table,
           W1, b1, g1, be1, rm1, rv1,
           W2, b2, g2, be2, rm2, rv2,
           W3, b3):
    B = user_id.shape[0]
    b_per_w = B // _NW

    uid = user_id.astype(jnp.int32)
    u_emb = _sc_gather(user_table.reshape(-1), uid, B, b_per_w).reshape(B, 64)

    s1 = g1 * lax.rsqrt(rv1 + EPS)
    W1f = W1 * s1[None, :]
    b1f = ((b1 - rm1) * s1 + be1)[None, :]
    s2 = g2 * lax.rsqrt(rv2 + EPS)
    W2f = W2 * s2[None, :]
    b2f = ((b2 - rm2) * s2 + be2)[None, :]

    lid2 = user_lang.astype(jnp.int32).reshape(B, 1)
    return _mlp(u_emb, user_continuous, lid2, lang_table,
                W1f[:64], W1f[64:80], W1f[80:83], b1f,
                W2f, b2f, W3, b3[None, :])


# speedup vs baseline: 1.0001x; 1.0001x over previous
import functools

import jax
import jax.numpy as jnp
from jax import lax
from jax.experimental import pallas as pl
from jax.experimental.pallas import tpu as pltpu
from jax.experimental.pallas import tpu_sc as plsc

EPS = 1e-5

_NW = 32
_CH = 32


def _sc_gather(utab, uid, B, b_per_w):
    mesh = plsc.VectorSubcoreMesh(core_axis_name="c", subcore_axis_name="s")

    @functools.partial(
        pl.kernel,
        mesh=mesh,
        compiler_params=pltpu.CompilerParams(needs_layout_passes=False),
        out_type=jax.ShapeDtypeStruct((B, 128), jnp.float32),
        scratch_types=[
            pltpu.VMEM((b_per_w,), jnp.int32),
            pltpu.VMEM((b_per_w, 128), jnp.float32),
            pltpu.SemaphoreType.DMA,
        ],
    )
    def k(tab, uid_h, out, uid_v, rows_v, sem):
        wid = lax.axis_index("s") * 2 + lax.axis_index("c")
        base = wid * b_per_w
        pltpu.sync_copy(uid_h.at[pl.ds(base, b_per_w)], uid_v)

        lanes = lax.iota(jnp.int32, 16)

        def body(g, carry):
            v = uid_v[pl.ds(g * 16, 16)]
            for l in range(16):
                rid = jnp.sum(jnp.where(lanes == l, v, 0))
                pair = lax.shift_right_logical(rid, 1)
                pltpu.async_copy(
                    tab.at[pl.ds(pair, 1)],
                    rows_v.at[pl.ds(g * 16 + l, 1)], sem)
            return carry

        lax.fori_loop(0, b_per_w // 16, body, 0)
        pltpu.make_async_copy(
            tab.at[pl.ds(0, b_per_w)], rows_v, sem).wait()
        pltpu.sync_copy(rows_v, out.at[pl.ds(base, b_per_w)])

    return k(utab, uid)


def _mlp_body(u_ref, c_ref, lid_ref, uid_ref, ltab_ref,
              w1u, w1l, w1c, b1, w2, b2, w3, b3, o_ref):
    u2 = u_ref[...]
    par = (uid_ref[...] & 1) == 1
    u = jnp.where(par, u2[:, 64:128], u2[:, 0:64])
    h = jnp.dot(u, w1u[...], preferred_element_type=jnp.float32)
    lp = jnp.dot(ltab_ref[...], w1l[...], preferred_element_type=jnp.float32)
    oh = (lid_ref[...] == lax.broadcasted_iota(
        jnp.int32, (lid_ref.shape[0], ltab_ref.shape[0]), 1)).astype(jnp.float32)
    h = h + jnp.dot(oh, lp, preferred_element_type=jnp.float32)
    h = h + jnp.dot(c_ref[...], w1c[...], preferred_element_type=jnp.float32)
    h = jnp.maximum(h + b1[...], 0.0)
    h = jnp.dot(h, w2[...], preferred_element_type=jnp.float32)
    h = jnp.maximum(h + b2[...], 0.0)
    o_ref[...] = jnp.dot(h, w3[...], preferred_element_type=jnp.float32) + b3[...]


def _mlp(u_emb, cont, lid2, uid2, lang_table, W1u, W1l, W1c, b1f, W2f, b2f,
         W3, b3, TB=2048):
    B = u_emb.shape[0]
    NL = lang_table.shape[0]
    grid = (B // TB,)
    full = lambda i: (0, 0)
    return pl.pallas_call(
        _mlp_body,
        grid=grid,
        in_specs=[
            pl.BlockSpec((TB, 128), lambda i: (i, 0)),
            pl.BlockSpec((TB, 3), lambda i: (i, 0)),
            pl.BlockSpec((TB, 1), lambda i: (i, 0)),
            pl.BlockSpec((TB, 1), lambda i: (i, 0)),
            pl.BlockSpec((NL, 16), full),
            pl.BlockSpec((64, 256), full),
            pl.BlockSpec((16, 256), full),
            pl.BlockSpec((3, 256), full),
            pl.BlockSpec((1, 256), full),
            pl.BlockSpec((256, 128), full),
            pl.BlockSpec((1, 128), full),
            pl.BlockSpec((128, 128), full),
            pl.BlockSpec((1, 128), full),
        ],
        out_specs=pl.BlockSpec((TB, 128), lambda i: (i, 0)),
        out_shape=jax.ShapeDtypeStruct((B, 128), jnp.float32),
    )(u_emb, cont, lid2, uid2, lang_table, W1u, W1l, W1c, b1f, W2f, b2f,
      W3, b3)


def kernel(user_id, user_continuous, user_lang, user_table, lang_table,
           W1, b1, g1, be1, rm1, rv1,
           W2, b2, g2, be2, rm2, rv2,
           W3, b3):
    B = user_id.shape[0]
    b_per_w = B // _NW

    uid = user_id.astype(jnp.int32)
    tab2 = user_table.reshape(user_table.shape[0] // 2, 128)
    u_emb = _sc_gather(tab2, uid, B, b_per_w)

    s1 = g1 * lax.rsqrt(rv1 + EPS)
    W1f = W1 * s1[None, :]
    b1f = ((b1 - rm1) * s1 + be1)[None, :]
    s2 = g2 * lax.rsqrt(rv2 + EPS)
    W2f = W2 * s2[None, :]
    b2f = ((b2 - rm2) * s2 + be2)[None, :]

    lid2 = user_lang.astype(jnp.int32).reshape(B, 1)
    uid2 = uid.reshape(B, 1)
    return _mlp(u_emb, user_continuous, lid2, uid2, lang_table,
                W1f[:64], W1f[64:80], W1f[80:83], b1f,
                W2f, b2f, W3, b3[None, :])


# speedup vs baseline: 1.7355x; 1.7353x over previous
import functools

import jax
import jax.numpy as jnp
from jax import lax
from jax.experimental import pallas as pl
from jax.experimental.pallas import tpu as pltpu
from jax.experimental.pallas import tpu_sc as plsc

EPS = 1e-5

_NW = 32
_CH = 64


def _sc_gather(tab3, uid, B, b_per_w):
    n_ch = b_per_w // _CH
    mesh = plsc.VectorSubcoreMesh(core_axis_name="c", subcore_axis_name="s")

    @functools.partial(
        pl.kernel,
        mesh=mesh,
        compiler_params=pltpu.CompilerParams(needs_layout_passes=False),
        out_type=jax.ShapeDtypeStruct((B, 8, 64), jnp.bfloat16),
        scratch_types=[
            pltpu.VMEM((b_per_w,), jnp.int32),
            pltpu.VMEM((_CH, 8, 64), jnp.bfloat16),
            pltpu.VMEM((_CH, 8, 64), jnp.bfloat16),
            pltpu.SemaphoreType.DMA,
            pltpu.SemaphoreType.DMA,
        ],
    )
    def k(tab, uid_h, out, uid_v, blk0, blk1, sem0, sem1):
        wid = lax.axis_index("s") * 2 + lax.axis_index("c")
        base = wid * b_per_w
        pltpu.sync_copy(uid_h.at[pl.ds(base, b_per_w)], uid_v)

        lanes = lax.iota(jnp.int32, 16)
        bufs = [(blk0, sem0), (blk1, sem1)]

        def fire(c, buf, sem):
            def body(g, carry):
                v = uid_v[pl.ds(c * _CH + g * 16, 16)]
                for l in range(16):
                    rid = jnp.sum(jnp.where(lanes == l, v, 0))
                    q = lax.shift_right_logical(rid, 3)
                    pltpu.async_copy(
                        tab.at[pl.ds(q, 1)],
                        buf.at[pl.ds(g * 16 + l, 1)], sem)
                return carry
            lax.fori_loop(0, _CH // 16, body, 0)

        def drain_and_out(c, buf, sem):
            pltpu.make_async_copy(tab.at[pl.ds(0, _CH)], buf, sem).wait()
            pltpu.sync_copy(buf, out.at[pl.ds(base + c * _CH, _CH)])

        fire(0, *bufs[0])
        fire(1, *bufs[1])
        for c in range(n_ch):
            drain_and_out(c, *bufs[c % 2])
            if c + 2 < n_ch:
                fire(c + 2, *bufs[c % 2])

    return k(tab3, uid)


def _mlp_body(u_ref, c_ref, lid_ref, uid_ref, ltab_ref,
              w1u, w1l, w1c, b1, w2, b2, w3, b3, o_ref):
    off = uid_ref[...] & 7
    blk = u_ref[...]
    u = jnp.zeros((blk.shape[0], 64), jnp.float32)
    for kk in range(8):
        u = u + jnp.where(off == kk, blk[:, kk, :].astype(jnp.float32), 0.0)
    h = jnp.dot(u, w1u[...], preferred_element_type=jnp.float32)
    lp = jnp.dot(ltab_ref[...], w1l[...], preferred_element_type=jnp.float32)
    oh = (lid_ref[...] == lax.broadcasted_iota(
        jnp.int32, (lid_ref.shape[0], ltab_ref.shape[0]), 1)).astype(jnp.float32)
    h = h + jnp.dot(oh, lp, preferred_element_type=jnp.float32)
    h = h + jnp.dot(c_ref[...], w1c[...], preferred_element_type=jnp.float32)
    h = jnp.maximum(h + b1[...], 0.0)
    h = jnp.dot(h, w2[...], preferred_element_type=jnp.float32)
    h = jnp.maximum(h + b2[...], 0.0)
    o_ref[...] = jnp.dot(h, w3[...], preferred_element_type=jnp.float32) + b3[...]


def _mlp(u_blk, cont, lid2, uid2, lang_table, W1u, W1l, W1c, b1f, W2f, b2f,
         W3, b3, TB=2048):
    B = cont.shape[0]
    NL = lang_table.shape[0]
    grid = (B // TB,)
    full = lambda i: (0, 0)
    return pl.pallas_call(
        _mlp_body,
        grid=grid,
        in_specs=[
            pl.BlockSpec((TB, 8, 64), lambda i: (i, 0, 0)),
            pl.BlockSpec((TB, 3), lambda i: (i, 0)),
            pl.BlockSpec((TB, 1), lambda i: (i, 0)),
            pl.BlockSpec((TB, 1), lambda i: (i, 0)),
            pl.BlockSpec((NL, 16), full),
            pl.BlockSpec((64, 256), full),
            pl.BlockSpec((16, 256), full),
            pl.BlockSpec((3, 256), full),
            pl.BlockSpec((1, 256), full),
            pl.BlockSpec((256, 128), full),
            pl.BlockSpec((1, 128), full),
            pl.BlockSpec((128, 128), full),
            pl.BlockSpec((1, 128), full),
        ],
        out_specs=pl.BlockSpec((TB, 128), lambda i: (i, 0)),
        out_shape=jax.ShapeDtypeStruct((B, 128), jnp.float32),
    )(u_blk, cont, lid2, uid2, lang_table, W1u, W1l, W1c, b1f, W2f, b2f,
      W3, b3)


def kernel(user_id, user_continuous, user_lang, user_table, lang_table,
           W1, b1, g1, be1, rm1, rv1,
           W2, b2, g2, be2, rm2, rv2,
           W3, b3):
    B = user_id.shape[0]
    b_per_w = B // _NW

    uid = user_id.astype(jnp.int32)
    tab3 = user_table.astype(jnp.bfloat16).reshape(
        user_table.shape[0] // 8, 8, 64)
    u_blk = _sc_gather(tab3, uid, B, b_per_w)

    s1 = g1 * lax.rsqrt(rv1 + EPS)
    W1f = W1 * s1[None, :]
    b1f = ((b1 - rm1) * s1 + be1)[None, :]
    s2 = g2 * lax.rsqrt(rv2 + EPS)
    W2f = W2 * s2[None, :]
    b2f = ((b2 - rm2) * s2 + be2)[None, :]

    lid2 = user_lang.astype(jnp.int32).reshape(B, 1)
    uid2 = uid.reshape(B, 1)
    return _mlp(u_blk, user_continuous, lid2, uid2, lang_table,
                W1f[:64], W1f[64:80], W1f[80:83], b1f,
                W2f, b2f, W3, b3[None, :])
